# baseline (speedup 1.0000x reference)
import functools

import jax
import jax.numpy as jnp
from jax import lax
from jax.experimental import pallas as pl
from jax.experimental.pallas import tpu as pltpu
from jax.experimental.pallas import tpu_sc as plsc

L = 16


@functools.lru_cache(maxsize=None)
def _build_sc_kernel(B, C, V, D):
    info = plsc.get_sparse_core_info()
    NC, NS = info.num_cores, info.num_subcores
    NW = NC * NS
    assert B % (NW * L) == 0
    BPW = B // NW
    CB = L
    NCH = BPW // CB
    DP = 2 * D
    DB = 8
    NDB = D // DB

    mesh = plsc.VectorSubcoreMesh(core_axis_name="c", subcore_axis_name="s")

    @functools.partial(
        pl.kernel,
        mesh=mesh,
        out_type=jax.ShapeDtypeStruct((B * C,), jnp.float32),
        compiler_params=pltpu.CompilerParams(
            needs_layout_passes=False,
            use_tc_tiling_on_sc=True,
        ),
        scratch_types=[
            pltpu.VMEM((BPW,), jnp.int32),
            pltpu.VMEM((BPW * C,), jnp.int32),
            pltpu.VMEM((BPW * C,), jnp.float32),
            pltpu.VMEM((CB, DP), jnp.float32),
            pltpu.VMEM((CB * C, DP), jnp.float32),
            pltpu.VMEM((CB, DP), jnp.float32),
            pltpu.VMEM((CB * C, DP), jnp.float32),
            pltpu.SemaphoreType.DMA,
            pltpu.SemaphoreType.DMA,
            pltpu.SemaphoreType.DMA,
            pltpu.SemaphoreType.DMA,
        ],
    )
    def sc_kernel(center_hbm, ctx_hbm, wc_hbm, wk_hbm, out_hbm,
                  cidx, kidx, outv,
                  crows0, krows0, crows1, krows1,
                  semc0, semk0, semc1, semk1):
        crows = (crows0, crows1)
        krows = (krows0, krows1)
        semc = (semc0, semc1)
        semk = (semk0, semk1)
        wid = lax.axis_index("s") * NC + lax.axis_index("c")
        wbase = wid * BPW
        iota = lax.broadcasted_iota(jnp.int32, (L,), 0)
        zerov = jnp.zeros((L,), jnp.float32)

        pltpu.sync_copy(center_hbm.at[pl.ds(wbase, BPW)], cidx)
        pltpu.sync_copy(ctx_hbm.at[pl.ds(wbase * C, BPW * C)], kidx)

        NSPL = 4
        SPL = CB * C // NSPL

        def dma_group(s, i):
            hs = [pltpu.make_async_copy(
                wc_hbm.at[cidx.at[pl.ds(i * CB, CB)]], crows[s], semc[s])]
            for j in range(NSPL):
                hs.append(pltpu.make_async_copy(
                    wk_hbm.at[kidx.at[pl.ds(i * CB * C + j * SPL, SPL)]],
                    krows[s].at[pl.ds(j * SPL, SPL)], semk[s]))
            return hs

        def issue(s, i):
            for h in dma_group(s, i):
                h.start()

        issue(0, jnp.int32(0))
        issue(1, jnp.int32(1))

        def chunk_pair_body(i2, carry):
            for s in range(2):
                i = i2 * 2 + s
                for h in dma_group(s, i):
                    h.wait()
                cr, kr = crows[s], krows[s]
                obase = i * (CB * C)

                def dblk_body(dblk, accs):
                    d0 = dblk * DB
                    cc = [
                        plsc.load_gather(cr, [iota, iota * 0 + (d0 + d)])
                        for d in range(DB)
                    ]
                    new_accs = []
                    for c in range(C):
                        a = accs[c]
                        rowc = iota * C + c
                        for d in range(DB):
                            kv = plsc.load_gather(
                                kr, [rowc, iota * 0 + (d0 + d)])
                            a = a + cc[d] * kv
                        new_accs.append(a)
                    return tuple(new_accs)

                accs = lax.fori_loop(0, NDB, dblk_body, (zerov,) * C)
                for c in range(C):
                    plsc.store_scatter(
                        outv, [iota * C + (obase + c)], accs[c])

                @pl.when(i + 2 < NCH)
                def _():
                    issue(s, i + 2)
            return carry

        lax.fori_loop(0, NCH // 2, chunk_pair_body, 0)
        pltpu.sync_copy(outv, out_hbm.at[pl.ds(wbase * C, BPW * C)])

    return sc_kernel


def kernel(center, context, W_center, W_context):
    B, C = context.shape
    V, D = W_center.shape
    center = jnp.asarray(center, jnp.int32)
    ctx_flat = jnp.asarray(context, jnp.int32).reshape(B * C)
    wc_p = jnp.pad(W_center, ((0, 0), (0, D)))
    wk_p = jnp.pad(W_context, ((0, 0), (0, D)))
    sc = _build_sc_kernel(B, C, V, D)
    out_flat = sc(center, ctx_flat, wc_p, wk_p)
    return out_flat.reshape(B, C)


# speedup vs baseline: 1.0057x; 1.0057x over previous
import functools

import jax
import jax.numpy as jnp
from jax import lax
from jax.experimental import pallas as pl
from jax.experimental.pallas import tpu as pltpu
from jax.experimental.pallas import tpu_sc as plsc

L = 16


@functools.lru_cache(maxsize=None)
def _build_sc_kernel(B, C, V, D):
    info = plsc.get_sparse_core_info()
    NC, NS = info.num_cores, info.num_subcores
    NW = NC * NS
    assert B % (NW * L) == 0
    BPW = B // NW
    CB = L
    NCH = BPW // CB
    DP = 2 * D
    DB = 8
    NDB = D // DB

    mesh = plsc.VectorSubcoreMesh(core_axis_name="c", subcore_axis_name="s")

    @functools.partial(
        pl.kernel,
        mesh=mesh,
        out_type=jax.ShapeDtypeStruct((B * C,), jnp.float32),
        compiler_params=pltpu.CompilerParams(
            needs_layout_passes=False,
            use_tc_tiling_on_sc=True,
        ),
        scratch_types=[
            pltpu.VMEM((BPW,), jnp.int32),
            pltpu.VMEM((BPW * C,), jnp.int32),
            pltpu.VMEM((BPW * C,), jnp.float32),
            pltpu.VMEM((CB, DP), jnp.float32),
            pltpu.VMEM((CB * C, DP), jnp.float32),
            pltpu.VMEM((CB, DP), jnp.float32),
            pltpu.VMEM((CB * C, DP), jnp.float32),
            pltpu.SemaphoreType.DMA,
            pltpu.SemaphoreType.DMA,
            pltpu.SemaphoreType.DMA,
            pltpu.SemaphoreType.DMA,
        ],
    )
    def sc_kernel(center_hbm, ctx_hbm, wc_hbm, wk_hbm, out_hbm,
                  cidx, kidx, outv,
                  crows0, krows0, crows1, krows1,
                  semc0, semk0, semc1, semk1):
        crows = (crows0, crows1)
        krows = (krows0, krows1)
        semc = (semc0, semc1)
        semk = (semk0, semk1)
        wid = lax.axis_index("s") * NC + lax.axis_index("c")
        wbase = wid * BPW
        iota = lax.broadcasted_iota(jnp.int32, (L,), 0)
        zerov = jnp.zeros((L,), jnp.float32)

        pltpu.sync_copy(center_hbm.at[pl.ds(wbase, BPW)], cidx)
        pltpu.sync_copy(ctx_hbm.at[pl.ds(wbase * C, BPW * C)], kidx)

        def dma_group(s, i):
            return [
                pltpu.make_async_copy(
                    wc_hbm.at[cidx.at[pl.ds(i * CB, CB)]], crows[s], semc[s]),
                pltpu.make_async_copy(
                    wk_hbm.at[kidx.at[pl.ds(i * CB * C, CB * C)]],
                    krows[s], semk[s]),
            ]

        def issue(s, i):
            for h in dma_group(s, i):
                h.start()

        issue(0, jnp.int32(0))
        issue(1, jnp.int32(1))

        def chunk_pair_body(i2, carry):
            for s in range(2):
                i = i2 * 2 + s
                for h in dma_group(s, i):
                    h.wait()
                cr, kr = crows[s], krows[s]
                obase = i * (CB * C)

                def dblk_body(dblk, accs):
                    d0 = dblk * DB
                    cc = [
                        plsc.load_gather(cr, [iota, iota * 0 + (d0 + d)])
                        for d in range(DB)
                    ]
                    new_accs = []
                    for c in range(C):
                        a = accs[c]
                        rowc = iota * C + c
                        for d in range(DB):
                            kv = plsc.load_gather(
                                kr, [rowc, iota * 0 + (d0 + d)])
                            a = a + cc[d] * kv
                        new_accs.append(a)
                    return tuple(new_accs)

                accs = lax.fori_loop(0, NDB, dblk_body, (zerov,) * C)
                for c in range(C):
                    plsc.store_scatter(
                        outv, [iota * C + (obase + c)], accs[c])

                @pl.when(i + 2 < NCH)
                def _():
                    issue(s, i + 2)
            return carry

        lax.fori_loop(0, NCH // 2, chunk_pair_body, 0)
        pltpu.sync_copy(outv, out_hbm.at[pl.ds(wbase * C, BPW * C)])

    return sc_kernel


def kernel(center, context, W_center, W_context):
    B, C = context.shape
    V, D = W_center.shape
    center = jnp.asarray(center, jnp.int32)
    ctx_flat = jnp.asarray(context, jnp.int32).reshape(B * C)
    wc_p = jnp.pad(W_center, ((0, 0), (0, D)))
    wk_p = jnp.pad(W_context, ((0, 0), (0, D)))
    sc = _build_sc_kernel(B, C, V, D)
    out_flat = sc(center, ctx_flat, wc_p, wk_p)
    return out_flat.reshape(B, C)
